# baseline (device time: 72739 ns/iter reference)
import jax
import jax.numpy as jnp
from jax import lax
from jax.experimental import pallas as pl
from jax.experimental.pallas import tpu as pltpu

N_DEV = 4
B, S, D = 2, 512, 2048
H, DH, DR = 16, 128, 32
DC = 512
DCS = DC // N_DEV
BS = B * S
HG = H // N_DEV
DG = HG * DH
N_COMM = 3

BF = jnp.bfloat16
F32 = jnp.float32


def _fused_body(x_ref, wdkv_ref, wuk_any, wuv_any, wkr_ref, wqr_any,
                wq_any, wo_any, out_ref,
                xbf_ref, c_ref, wukbf_ref, wuvbf_ref, wuk_sl, wuv_sl,
                obuf_ref, wq_st, wo_st,
                wuk_ref, wuv_ref, wqr_ref,
                p1_send, p1_recv, o_send, o_recv, wq_sem, wo_sem, in_sems):
    my = lax.axis_index("i")

    in_cps = []
    for i, (src, dst) in enumerate((
            (wuk_any, wuk_ref), (wuv_any, wuv_ref), (wqr_any, wqr_ref))):
        cp = pltpu.make_async_copy(src, dst, in_sems.at[i])
        cp.start()
        in_cps.append(cp)

    barrier = pltpu.get_barrier_semaphore()
    for p in range(1, N_DEV):
        pl.semaphore_signal(barrier, inc=1,
                            device_id=(lax.rem(my + p, N_DEV),),
                            device_id_type=pl.DeviceIdType.MESH)

    wq_cp = pltpu.make_async_copy(
        wq_any.at[:, pl.ds(my * DG, DG)], wq_st, wq_sem)
    wq_cp.start()
    wo_cp = pltpu.make_async_copy(
        wo_any.at[pl.ds(my * DG, DG), :], wo_st, wo_sem)
    wo_cp.start()

    for b in range(B):
        xbf_ref[b * S:(b + 1) * S, :] = x_ref[b].astype(BF)
    xbf = xbf_ref[...]

    c_ref[my] = jnp.dot(xbf, wdkv_ref[...].astype(BF),
                        preferred_element_type=F32).astype(BF)
    in_cps[0].wait()
    in_cps[1].wait()
    wukbf_ref[...] = wuk_ref[...].astype(BF)
    wuvbf_ref[...] = wuv_ref[...].astype(BF)
    wuk_sl[my] = wukbf_ref[:, pl.ds(my * DG, DG)]
    wuv_sl[my] = wuvbf_ref[:, pl.ds(my * DG, DG)]

    pl.semaphore_wait(barrier, N_DEV - 1)
    p1 = []
    for p in range(1, N_DEV):
        dst = lax.rem(my + p, N_DEV)
        for t, (src, dref) in enumerate((
                (c_ref.at[my], c_ref.at[my]),
                (wukbf_ref.at[:, pl.ds(dst * DG, DG)], wuk_sl.at[my]),
                (wuvbf_ref.at[:, pl.ds(dst * DG, DG)], wuv_sl.at[my]))):
            rdma = pltpu.make_async_remote_copy(
                src_ref=src, dst_ref=dref,
                send_sem=p1_send.at[p - 1, t],
                recv_sem=p1_recv.at[p - 1, t],
                device_id=(dst,),
                device_id_type=pl.DeviceIdType.MESH,
            )
            rdma.start()
            p1.append(rdma)

    in_cps[2].wait()
    kr = jnp.dot(xbf, wkr_ref[...].astype(BF),
                 preferred_element_type=F32).astype(BF)
    wqr_my = wqr_ref[:, pl.ds(my * HG * DR, HG * DR)].astype(BF)
    qr_my = jnp.dot(xbf, wqr_my, preferred_element_type=F32).astype(BF)
    wq_cp.wait()
    q_my = jnp.dot(xbf, wq_st[...].astype(BF),
                   preferred_element_type=F32).astype(BF)

    k_acc = jnp.dot(c_ref[my], wuk_sl[my], preferred_element_type=F32)
    v_acc = jnp.dot(c_ref[my], wuv_sl[my], preferred_element_type=F32)
    for p in range(1, N_DEV):
        for t in range(N_COMM):
            p1[(p - 1) * N_COMM + t].wait_recv()
        org = lax.rem(my + N_DEV - p, N_DEV)
        k_acc = k_acc + jnp.dot(c_ref[org], wuk_sl[org],
                                preferred_element_type=F32)
        v_acc = v_acc + jnp.dot(c_ref[org], wuv_sl[org],
                                preferred_element_type=F32)
    k_my = k_acc.astype(BF)
    v_my = v_acc.astype(BF)

    scale = (DH + DR) ** -0.5
    nt = (((1,), (1,)), ((), ()))
    o_rdmas = []
    for hh in range(HG):
        ds_h = slice(hh * DH, (hh + 1) * DH)
        qh = jnp.concatenate([q_my[:, ds_h], qr_my[:, hh * DR:(hh + 1) * DR]],
                             axis=1)
        kh = jnp.concatenate([k_my[:, ds_h], kr], axis=1)
        vh = v_my[:, ds_h]
        for b in range(B):
            sl = slice(b * S, (b + 1) * S)
            s = lax.dot_general(qh[sl], kh[sl], nt,
                                preferred_element_type=F32)
            p = jnp.exp(s * scale)
            denom = jnp.sum(p, axis=-1, keepdims=True)
            o_b = jnp.dot(p.astype(BF), vh[sl], preferred_element_type=F32)
            obuf_ref[my, sl, ds_h] = (o_b * (1.0 / denom)).astype(BF)
        stripe = pl.ds(hh * DH, DH)
        for p in range(1, N_DEV):
            dst = lax.rem(my + p, N_DEV)
            rdma = pltpu.make_async_remote_copy(
                src_ref=obuf_ref.at[my, :, stripe],
                dst_ref=obuf_ref.at[my, :, stripe],
                send_sem=o_send.at[p - 1, hh], recv_sem=o_recv.at[p - 1, hh],
                device_id=(dst,), device_id_type=pl.DeviceIdType.MESH,
            )
            rdma.start()
            o_rdmas.append(rdma)

    def proj(org, first):
        wo_cp.wait()
        wo_bf = wo_st[...].astype(BF)
        ob = obuf_ref[org]
        for j in range(2):
            half = pl.ds(j * (D // 2), D // 2)
            prod = jnp.dot(ob, wo_bf[:, j * (D // 2):(j + 1) * (D // 2)],
                           preferred_element_type=F32).reshape(B, S, D // 2)
            if first:
                out_ref[:, :, half] = prod
            else:
                out_ref[:, :, half] = out_ref[:, :, half] + prod

    proj(my, True)
    for p in range(1, N_DEV):
        org = lax.rem(my + N_DEV - p, N_DEV)
        cp = pltpu.make_async_copy(
            wo_any.at[pl.ds(org * DG, DG), :], wo_st, wo_sem)
        cp.start()
        for hh in range(HG):
            o_rdmas[hh * (N_DEV - 1) + (p - 1)].wait_recv()
        proj(org, False)

    for rdma in p1 + o_rdmas:
        rdma.wait_send()


def kernel(x, Wdkv, Wuk, Wuv, Wq, Wqr, Wkr, Wo):
    return pl.pallas_call(
        _fused_body,
        out_shape=jax.ShapeDtypeStruct((B, S, D), F32),
        in_specs=[
            pl.BlockSpec(memory_space=pltpu.VMEM),
            pl.BlockSpec(memory_space=pltpu.VMEM),
            pl.BlockSpec(memory_space=pl.ANY),
            pl.BlockSpec(memory_space=pl.ANY),
            pl.BlockSpec(memory_space=pltpu.VMEM),
            pl.BlockSpec(memory_space=pl.ANY),
            pl.BlockSpec(memory_space=pl.ANY),
            pl.BlockSpec(memory_space=pl.ANY),
        ],
        out_specs=pl.BlockSpec(memory_space=pltpu.VMEM),
        scratch_shapes=[
            pltpu.VMEM((BS, D), BF),
            pltpu.VMEM((N_DEV, BS, DCS), BF),
            pltpu.VMEM((DCS, D), BF),
            pltpu.VMEM((DCS, D), BF),
            pltpu.VMEM((N_DEV, DCS, DG), BF),
            pltpu.VMEM((N_DEV, DCS, DG), BF),
            pltpu.VMEM((N_DEV, BS, DG), BF),
            pltpu.VMEM((D, DG), F32),
            pltpu.VMEM((DG, D), F32),
            pltpu.VMEM((DCS, D), F32),
            pltpu.VMEM((DCS, D), F32),
            pltpu.VMEM((D, DC), F32),
            pltpu.SemaphoreType.DMA((N_DEV - 1, N_COMM)),
            pltpu.SemaphoreType.DMA((N_DEV - 1, N_COMM)),
            pltpu.SemaphoreType.DMA((N_DEV - 1, HG)),
            pltpu.SemaphoreType.DMA((N_DEV - 1, HG)),
            pltpu.SemaphoreType.DMA,
            pltpu.SemaphoreType.DMA,
            pltpu.SemaphoreType.DMA((3,)),
        ],
        compiler_params=pltpu.CompilerParams(
            collective_id=0,
            vmem_limit_bytes=52 * 1024 * 1024,
        ),
    )(x, Wdkv, Wuk, Wuv, Wkr, Wqr, Wq, Wo)


# device time: 69676 ns/iter; 1.0440x vs baseline; 1.0440x over previous
import jax
import jax.numpy as jnp
from jax import lax
from jax.experimental import pallas as pl
from jax.experimental.pallas import tpu as pltpu

N_DEV = 4
B, S, D = 2, 512, 2048
H, DH, DR = 16, 128, 32
DC = 512
DCS = DC // N_DEV
BS = B * S
HG = H // N_DEV
DG = HG * DH
N_COMM = 3

BF = jnp.bfloat16
F32 = jnp.float32


def _fused_body(x_ref, wdkv_ref, wuk_ref, wuv_ref, wkr_ref, wqr_ref,
                wq_any, wo_any, out_ref,
                xbf_ref, c_ref, wukbf_ref, wuvbf_ref, wuk_sl, wuv_sl,
                obuf_ref, wq_st, wo_st,
                p1_send, p1_recv, o_send, o_recv, wq_sem, wo_sem):
    my = lax.axis_index("i")

    barrier = pltpu.get_barrier_semaphore()
    for p in range(1, N_DEV):
        pl.semaphore_signal(barrier, inc=1,
                            device_id=(lax.rem(my + p, N_DEV),),
                            device_id_type=pl.DeviceIdType.MESH)

    wq_cp = pltpu.make_async_copy(
        wq_any.at[:, pl.ds(my * DG, DG)], wq_st, wq_sem)
    wq_cp.start()
    wo_cp = pltpu.make_async_copy(
        wo_any.at[pl.ds(my * DG, DG), :], wo_st, wo_sem)
    wo_cp.start()

    for b in range(B):
        xbf_ref[b * S:(b + 1) * S, :] = x_ref[b].astype(BF)
    xbf = xbf_ref[...]

    c_ref[my] = jnp.dot(xbf, wdkv_ref[...].astype(BF),
                        preferred_element_type=F32).astype(BF)
    wukbf_ref[...] = wuk_ref[...].astype(BF)
    wuvbf_ref[...] = wuv_ref[...].astype(BF)
    wuk_sl[my] = wukbf_ref[:, pl.ds(my * DG, DG)]
    wuv_sl[my] = wuvbf_ref[:, pl.ds(my * DG, DG)]

    pl.semaphore_wait(barrier, N_DEV - 1)
    p1 = []
    for p in range(1, N_DEV):
        dst = lax.rem(my + p, N_DEV)
        for t, (src, dref) in enumerate((
                (c_ref.at[my], c_ref.at[my]),
                (wukbf_ref.at[:, pl.ds(dst * DG, DG)], wuk_sl.at[my]),
                (wuvbf_ref.at[:, pl.ds(dst * DG, DG)], wuv_sl.at[my]))):
            rdma = pltpu.make_async_remote_copy(
                src_ref=src, dst_ref=dref,
                send_sem=p1_send.at[p - 1, t],
                recv_sem=p1_recv.at[p - 1, t],
                device_id=(dst,),
                device_id_type=pl.DeviceIdType.MESH,
            )
            rdma.start()
            p1.append(rdma)

    kr = jnp.dot(xbf, wkr_ref[...].astype(BF),
                 preferred_element_type=F32).astype(BF)
    wqr_my = wqr_ref[:, pl.ds(my * HG * DR, HG * DR)].astype(BF)
    qr_my = jnp.dot(xbf, wqr_my, preferred_element_type=F32).astype(BF)
    wq_cp.wait()
    q_my = jnp.dot(xbf, wq_st[...].astype(BF),
                   preferred_element_type=F32).astype(BF)

    k_acc = jnp.dot(c_ref[my], wuk_sl[my], preferred_element_type=F32)
    v_acc = jnp.dot(c_ref[my], wuv_sl[my], preferred_element_type=F32)
    for p in range(1, N_DEV):
        for t in range(N_COMM):
            p1[(p - 1) * N_COMM + t].wait_recv()
        org = lax.rem(my + N_DEV - p, N_DEV)
        k_acc = k_acc + jnp.dot(c_ref[org], wuk_sl[org],
                                preferred_element_type=F32)
        v_acc = v_acc + jnp.dot(c_ref[org], wuv_sl[org],
                                preferred_element_type=F32)
    k_my = k_acc.astype(BF)
    v_my = v_acc.astype(BF)

    scale = (DH + DR) ** -0.5
    nt = (((1,), (1,)), ((), ()))
    o_rdmas = []
    for hh in range(HG):
        ds_h = slice(hh * DH, (hh + 1) * DH)
        qh = jnp.concatenate([q_my[:, ds_h], qr_my[:, hh * DR:(hh + 1) * DR]],
                             axis=1)
        kh = jnp.concatenate([k_my[:, ds_h], kr], axis=1)
        vh = v_my[:, ds_h]
        for b in range(B):
            sl = slice(b * S, (b + 1) * S)
            s = lax.dot_general(qh[sl], kh[sl], nt,
                                preferred_element_type=F32)
            p = jnp.exp(s * scale)
            denom = jnp.sum(p, axis=-1, keepdims=True)
            o_b = jnp.dot(p.astype(BF), vh[sl], preferred_element_type=F32)
            obuf_ref[my, sl, ds_h] = (o_b * (1.0 / denom)).astype(BF)
        stripe = pl.ds(hh * DH, DH)
        for p in range(1, N_DEV):
            dst = lax.rem(my + p, N_DEV)
            rdma = pltpu.make_async_remote_copy(
                src_ref=obuf_ref.at[my, :, stripe],
                dst_ref=obuf_ref.at[my, :, stripe],
                send_sem=o_send.at[p - 1, hh], recv_sem=o_recv.at[p - 1, hh],
                device_id=(dst,), device_id_type=pl.DeviceIdType.MESH,
            )
            rdma.start()
            o_rdmas.append(rdma)

    def proj(org, first):
        wo_cp.wait()
        wo_bf = wo_st[...].astype(BF)
        ob = obuf_ref[org]
        for j in range(2):
            half = pl.ds(j * (D // 2), D // 2)
            prod = jnp.dot(ob, wo_bf[:, j * (D // 2):(j + 1) * (D // 2)],
                           preferred_element_type=F32).reshape(B, S, D // 2)
            if first:
                out_ref[:, :, half] = prod
            else:
                out_ref[:, :, half] = out_ref[:, :, half] + prod

    proj(my, True)
    for p in range(1, N_DEV):
        org = lax.rem(my + N_DEV - p, N_DEV)
        cp = pltpu.make_async_copy(
            wo_any.at[pl.ds(org * DG, DG), :], wo_st, wo_sem)
        cp.start()
        for hh in range(HG):
            o_rdmas[hh * (N_DEV - 1) + (p - 1)].wait_recv()
        proj(org, False)

    for rdma in p1 + o_rdmas:
        rdma.wait_send()


def kernel(x, Wdkv, Wuk, Wuv, Wq, Wqr, Wkr, Wo):
    return pl.pallas_call(
        _fused_body,
        out_shape=jax.ShapeDtypeStruct((B, S, D), F32),
        in_specs=[pl.BlockSpec(memory_space=pltpu.VMEM)] * 6
        + [pl.BlockSpec(memory_space=pl.ANY)] * 2,
        out_specs=pl.BlockSpec(memory_space=pltpu.VMEM),
        scratch_shapes=[
            pltpu.VMEM((BS, D), BF),
            pltpu.VMEM((N_DEV, BS, DCS), BF),
            pltpu.VMEM((DCS, D), BF),
            pltpu.VMEM((DCS, D), BF),
            pltpu.VMEM((N_DEV, DCS, DG), BF),
            pltpu.VMEM((N_DEV, DCS, DG), BF),
            pltpu.VMEM((N_DEV, BS, DG), BF),
            pltpu.VMEM((D, DG), F32),
            pltpu.VMEM((DG, D), F32),
            pltpu.SemaphoreType.DMA((N_DEV - 1, N_COMM)),
            pltpu.SemaphoreType.DMA((N_DEV - 1, N_COMM)),
            pltpu.SemaphoreType.DMA((N_DEV - 1, HG)),
            pltpu.SemaphoreType.DMA((N_DEV - 1, HG)),
            pltpu.SemaphoreType.DMA,
            pltpu.SemaphoreType.DMA,
        ],
        compiler_params=pltpu.CompilerParams(collective_id=0),
    )(x, Wdkv, Wuk, Wuv, Wkr, Wqr, Wq, Wo)
